# Initial kernel scaffold; baseline (speedup 1.0000x reference)
import jax
import jax.numpy as jnp
from jax.experimental import pallas as pl


def kernel(nodes, edges, globals_, senders, receivers, ew0, eb0, ew1, eb1, ew2, eb2, nw0, nb0, nw1, nb1, nw2, nb2, gw0, gb0, gw1, gb1, gw2, gb2):
    raise NotImplementedError("write your pallas kernel here")


# speedup vs baseline: 7.6442x; 7.6442x over previous
import functools

import jax
import jax.numpy as jnp
from jax import lax
from jax.experimental import pallas as pl
from jax.experimental.pallas import tpu as pltpu
from jax.experimental.pallas import tpu_sc as plsc

N = 50000
E = 800000
D = 16
NPASS = 3

NPAD = 50048
NPT = NPAD // 16
E8 = E // 8
N8 = N // 8
N8P = NPAD // 8
NCHUNK = E // 128
NW = 32
CH_PER_W = NCHUNK // NW
CH_EXTRA = NCHUNK - CH_PER_W * NW
CH_SUPER = (CH_PER_W // 4) * 4

BLK_E = 2000

_f32 = jnp.float32


def _worker_chunks():
    cid = lax.axis_index("c")
    sid = lax.axis_index("s")
    wid = sid * 2 + cid
    base = wid * CH_PER_W + jnp.minimum(wid, CH_EXTRA)
    n_ch = CH_PER_W + (wid < CH_EXTRA).astype(jnp.int32)
    return cid, sid, base, n_ch


def _gather_body(nodes_hbm, send1d, recv1d, out_s, out_r,
                 tab, idx_s, idx_r, rows_s, rows_r, sem, semo):
    cid, sid, base, n_ch = _worker_chunks()
    pltpu.sync_copy(nodes_hbm.at[pl.ds(sid * NPT, NPT)],
                    tab.at[pl.ds(sid * NPT, NPT)])
    plsc.subcore_barrier()

    def super_body(k, carry):
        e0 = (base + k * 4) * 128
        pltpu.sync_copy(send1d.at[pl.ds(e0, 512)], idx_s)
        pltpu.sync_copy(recv1d.at[pl.ds(e0, 512)], idx_r)
        cps = []
        for j in range(4):
            sl = pl.ds(j * 128, 128)
            cps.append(pltpu.async_copy(tab.at[idx_s.at[sl]],
                                        rows_s.at[sl], sem))
            cps.append(pltpu.async_copy(tab.at[idx_r.at[sl]],
                                        rows_r.at[sl], sem))
        for cp in cps:
            cp.wait()
        pltpu.async_copy(rows_s, out_s.at[pl.ds(e0, 512)], semo).wait()
        pltpu.async_copy(rows_r, out_r.at[pl.ds(e0, 512)], semo).wait()
        return carry

    lax.fori_loop(0, CH_PER_W // 4, super_body, 0)

    def tail_body(t, carry):
        e0 = (base + CH_SUPER + t) * 128
        sl = pl.ds(0, 128)
        pltpu.sync_copy(send1d.at[pl.ds(e0, 128)], idx_s.at[sl])
        pltpu.sync_copy(recv1d.at[pl.ds(e0, 128)], idx_r.at[sl])
        c1 = pltpu.async_copy(tab.at[idx_s.at[sl]], rows_s.at[sl], sem)
        c2 = pltpu.async_copy(tab.at[idx_r.at[sl]], rows_r.at[sl], sem)
        c1.wait()
        c2.wait()
        pltpu.async_copy(rows_s.at[sl], out_s.at[pl.ds(e0, 128)], semo).wait()
        pltpu.async_copy(rows_r.at[sl], out_r.at[pl.ds(e0, 128)], semo).wait()
        return carry

    lax.fori_loop(0, n_ch - CH_SUPER, tail_body, 0)


def _make_gather():
    mesh = plsc.VectorSubcoreMesh(core_axis_name="c", subcore_axis_name="s")
    return pl.kernel(
        _gather_body,
        mesh=mesh,
        compiler_params=pltpu.CompilerParams(use_tc_tiling_on_sc=False),
        out_type=[jax.ShapeDtypeStruct((E, D), _f32),
                  jax.ShapeDtypeStruct((E, D), _f32)],
        scratch_types=[
            pltpu.VMEM_SHARED((NPAD, D), _f32),
            pltpu.VMEM((512,), jnp.int32),
            pltpu.VMEM((512,), jnp.int32),
            pltpu.VMEM((512, D), _f32),
            pltpu.VMEM((512, D), _f32),
            pltpu.SemaphoreType.DMA,
            pltpu.SemaphoreType.DMA,
        ],
    )


def _zero_my_slice(acc, zbuf, sid):
    def zrow(i, carry):
        zbuf[i] = jnp.zeros((D,), _f32)
        return carry

    lax.fori_loop(0, 136, zrow, 0)

    def zslice(j, carry):
        pltpu.sync_copy(zbuf, acc.at[pl.ds(sid * NPT + j * 136, 136)])
        return carry

    lax.fori_loop(0, NPT // 136, zslice, 0)


def _scatter_body(edges_hbm, recv1d, part_out, acc, data, idx, zbuf, sem):
    cid, sid, base, n_ch = _worker_chunks()
    _zero_my_slice(acc, zbuf, sid)
    plsc.subcore_barrier()

    def body(c, carry):
        e0 = (base + c) * 128
        pltpu.sync_copy(edges_hbm.at[pl.ds(e0, 128)], data)
        pltpu.sync_copy(recv1d.at[pl.ds(e0, 128)], idx)
        pltpu.sync_copy(data, acc.at[idx], add=True)
        return carry

    lax.fori_loop(0, n_ch, body, 0)
    plsc.subcore_barrier()
    pltpu.sync_copy(acc.at[pl.ds(sid * NPT, NPT)],
                    part_out.at[cid, pl.ds(sid * NPT, NPT)])


def _make_scatter():
    mesh = plsc.VectorSubcoreMesh(core_axis_name="c", subcore_axis_name="s")
    return pl.kernel(
        _scatter_body,
        mesh=mesh,
        compiler_params=pltpu.CompilerParams(use_tc_tiling_on_sc=False),
        out_type=jax.ShapeDtypeStruct((2, NPAD, D), _f32),
        scratch_types=[
            pltpu.VMEM_SHARED((NPAD, D), _f32),
            pltpu.VMEM((128, D), _f32),
            pltpu.VMEM((128,), jnp.int32),
            pltpu.VMEM((136, D), _f32),
            pltpu.SemaphoreType.DMA,
        ],
    )


def _count_body(recv1d, part_out, acc, ones, idx, zbuf, sem):
    cid, sid, base, n_ch = _worker_chunks()

    def orow(i, carry):
        ones[i] = jnp.full((D,), 1.0, _f32)
        return carry

    lax.fori_loop(0, 128, orow, 0)
    _zero_my_slice(acc, zbuf, sid)
    plsc.subcore_barrier()

    def body(c, carry):
        e0 = (base + c) * 128
        pltpu.sync_copy(recv1d.at[pl.ds(e0, 128)], idx)
        pltpu.sync_copy(ones, acc.at[idx], add=True)
        return carry

    lax.fori_loop(0, n_ch, body, 0)
    plsc.subcore_barrier()
    pltpu.sync_copy(acc.at[pl.ds(sid * NPT, NPT)],
                    part_out.at[cid, pl.ds(sid * NPT, NPT)])


def _make_count():
    mesh = plsc.VectorSubcoreMesh(core_axis_name="c", subcore_axis_name="s")
    return pl.kernel(
        _count_body,
        mesh=mesh,
        compiler_params=pltpu.CompilerParams(use_tc_tiling_on_sc=False),
        out_type=jax.ShapeDtypeStruct((2, NPAD, D), _f32),
        scratch_types=[
            pltpu.VMEM_SHARED((NPAD, D), _f32),
            pltpu.VMEM((128, D), _f32),
            pltpu.VMEM((128,), jnp.int32),
            pltpu.VMEM((136, D), _f32),
            pltpu.SemaphoreType.DMA,
        ],
    )


def _edge_body(xe, xr, xs, g, w0gt, b0t, w0e, w0r, w0s, w1, b1t, w2, b2t,
               yout, esum, acc):
    i = pl.program_id(0)
    gb = jnp.dot(g[...], w0gt[...], preferred_element_type=_f32) + b0t[...]
    h = (jnp.dot(xe[...], w0e[...], preferred_element_type=_f32)
         + jnp.dot(xr[...], w0r[...], preferred_element_type=_f32)
         + jnp.dot(xs[...], w0s[...], preferred_element_type=_f32)
         + gb)
    h = jnp.maximum(h, 0.0)
    h = jnp.maximum(
        jnp.dot(h, w1[...], preferred_element_type=_f32) + b1t[...], 0.0)
    y = jnp.dot(h, w2[...], preferred_element_type=_f32) + b2t[...]
    yout[...] = y

    @pl.when(i == 0)
    def _():
        acc[...] = jnp.zeros_like(acc)

    acc[...] += jnp.sum(y, axis=0, keepdims=True)

    @pl.when(i == pl.num_programs(0) - 1)
    def _():
        esum[...] = acc[...]


def _edge_call(xe8, xr8, xs8, g, w0gt, b0t, w0e, w0r, w0s, w1, b1t, w2, b2t):
    blk = pl.BlockSpec((BLK_E, 128), lambda i: (i, 0))
    rep = lambda shape: pl.BlockSpec(shape, lambda i: (0, 0))
    return pl.pallas_call(
        _edge_body,
        grid=(E8 // BLK_E,),
        in_specs=[blk, blk, blk, rep((1, D)), rep((D, 128)), rep((1, 128)),
                  rep((128, 128)), rep((128, 128)), rep((128, 128)),
                  rep((128, 128)), rep((1, 128)),
                  rep((128, 128)), rep((1, 128))],
        out_specs=[blk, rep((1, 128))],
        out_shape=[jax.ShapeDtypeStruct((E8, 128), _f32),
                   jax.ShapeDtypeStruct((1, 128), _f32)],
        scratch_shapes=[pltpu.VMEM((1, 128), _f32)],
    )(xe8, xr8, xs8, g, w0gt, b0t, w0e, w0r, w0s, w1, b1t, w2, b2t)


def _node_body(p0, p1, c0, c1, xn, g, nw0gt, nb0t, nw0a, nw0b, nw1, nb1t,
               nw2, nb2t, esum, gw0e, gw0n, gw0g, gb0, gw1, gb1, gw2, gb2,
               yout, gout):
    cnt = c0[...] + c1[...]
    s = p0[...] + p1[...]
    inv = jnp.where(cnt > 0, 1.0 / jnp.maximum(cnt, 1.0), 0.0)
    agg = s * inv
    gb = jnp.dot(g[...], nw0gt[...], preferred_element_type=_f32) + nb0t[...]
    h = (jnp.dot(agg, nw0a[...], preferred_element_type=_f32)
         + jnp.dot(xn[...], nw0b[...], preferred_element_type=_f32)
         + gb)
    h = jnp.maximum(h, 0.0)
    h = jnp.maximum(
        jnp.dot(h, nw1[...], preferred_element_type=_f32) + nb1t[...], 0.0)
    y = jnp.dot(h, nw2[...], preferred_element_type=_f32) + nb2t[...]
    yout[...] = y

    ns = jnp.sum(y[0:N8, :], axis=0, keepdims=True)
    es = esum[...]
    e16 = jnp.zeros((1, D), _f32)
    n16 = jnp.zeros((1, D), _f32)
    for k in range(8):
        e16 = e16 + es[:, k * D:(k + 1) * D]
        n16 = n16 + ns[:, k * D:(k + 1) * D]
    mean_e = e16 * (1.0 / E)
    mean_n = n16 * (1.0 / N)
    gh = (jnp.dot(mean_e, gw0e[...], preferred_element_type=_f32)
          + jnp.dot(mean_n, gw0n[...], preferred_element_type=_f32)
          + jnp.dot(g[...], gw0g[...], preferred_element_type=_f32)
          + gb0[...])
    gh = jnp.maximum(gh, 0.0)
    gh = jnp.maximum(
        jnp.dot(gh, gw1[...], preferred_element_type=_f32) + gb1[...], 0.0)
    gout[...] = jnp.dot(gh, gw2[...], preferred_element_type=_f32) + gb2[...]


def _node_call(p0, p1, c0, c1, xn8, g, nw0gt, nb0t, nw0a, nw0b, nw1, nb1t,
               nw2, nb2t, esum, gw0e, gw0n, gw0g, gb0, gw1, gb1, gw2, gb2):
    full = lambda shape: pl.BlockSpec(shape, lambda: (0, 0))
    sds = jax.ShapeDtypeStruct
    return pl.pallas_call(
        _node_body,
        in_specs=[full((N8P, 128)), full((N8P, 128)), full((N8P, 128)),
                  full((N8P, 128)), full((N8P, 128)), full((1, D)),
                  full((D, 128)), full((1, 128)), full((128, 128)),
                  full((128, 128)), full((128, 128)), full((1, 128)),
                  full((128, 128)), full((1, 128)), full((1, 128)),
                  full((D, D)), full((D, D)), full((D, D)), full((1, D)),
                  full((D, D)), full((1, D)), full((D, D)), full((1, D))],
        out_specs=[full((N8P, 128)), full((1, D))],
        out_shape=[sds((N8P, 128), _f32), sds((1, D), _f32)],
    )(p0, p1, c0, c1, xn8, g, nw0gt, nb0t, nw0a, nw0b, nw1, nb1t, nw2, nb2t,
      esum, gw0e, gw0n, gw0g, gb0, gw1, gb1, gw2, gb2)


def kernel(nodes, edges, globals_, senders, receivers,
           ew0, eb0, ew1, eb1, ew2, eb2,
           nw0, nb0, nw1, nb1, nw2, nb2,
           gw0, gb0, gw1, gb1, gw2, gb2):
    send1d = senders.astype(jnp.int32)
    recv1d = receivers.astype(jnp.int32)

    i8 = jnp.eye(8, dtype=_f32)
    w0e = jnp.kron(i8, ew0[0:16])
    w0r = jnp.kron(i8, ew0[16:32])
    w0s = jnp.kron(i8, ew0[32:48])
    w0gt = jnp.tile(ew0[48:64], (1, 8))
    b0t = jnp.tile(eb0[None, :], (1, 8))
    w1k = jnp.kron(i8, ew1)
    b1t = jnp.tile(eb1[None, :], (1, 8))
    w2k = jnp.kron(i8, ew2)
    b2t = jnp.tile(eb2[None, :], (1, 8))

    nw0a = jnp.kron(i8, nw0[0:16])
    nw0b = jnp.kron(i8, nw0[16:32])
    nw0gt = jnp.tile(nw0[32:48], (1, 8))
    nb0t = jnp.tile(nb0[None, :], (1, 8))
    nw1k = jnp.kron(i8, nw1)
    nb1t = jnp.tile(nb1[None, :], (1, 8))
    nw2k = jnp.kron(i8, nw2)
    nb2t = jnp.tile(nb2[None, :], (1, 8))

    gw0e = gw0[0:16]
    gw0n = gw0[16:32]
    gw0g = gw0[32:48]
    gb0r = gb0[None, :]
    gb1r = gb1[None, :]
    gb2r = gb2[None, :]

    gather = _make_gather()
    scatter = _make_scatter()
    count = _make_count()

    cpart = count(recv1d)
    c0 = cpart[0].reshape(N8P, 128)
    c1 = cpart[1].reshape(N8P, 128)

    edges8 = edges.reshape(E8, 128)
    nodes_pad = jnp.concatenate(
        [nodes, jnp.zeros((NPAD - N, D), _f32)], axis=0)
    g_cur = globals_
    for _ in range(NPASS):
        out_s, out_r = gather(nodes_pad, send1d, recv1d)
        edges8, esum = _edge_call(
            edges8, out_r.reshape(E8, 128), out_s.reshape(E8, 128), g_cur,
            w0gt, b0t, w0e, w0r, w0s, w1k, b1t, w2k, b2t)
        part = scatter(edges8.reshape(E, D), recv1d)
        nodes8, g_cur = _node_call(
            part[0].reshape(N8P, 128), part[1].reshape(N8P, 128), c0, c1,
            nodes_pad.reshape(N8P, 128), g_cur, nw0gt, nb0t, nw0a, nw0b,
            nw1k, nb1t, nw2k, nb2t, esum,
            gw0e, gw0n, gw0g, gb0r, gw1, gb1r, gw2, gb2r)
        nodes_pad = nodes8.reshape(NPAD, D)

    return (nodes_pad[:N], edges8.reshape(E, D), g_cur)


# speedup vs baseline: 10.3727x; 1.3569x over previous
import functools

import jax
import jax.numpy as jnp
from jax import lax
from jax.experimental import pallas as pl
from jax.experimental.pallas import tpu as pltpu
from jax.experimental.pallas import tpu_sc as plsc

N = 50000
E = 800000
D = 16
NPASS = 3

NPAD = 50048
NPT = NPAD // 16
E8 = E // 8
N8 = N // 8
N8P = NPAD // 8
NCHUNK = E // 128
NW = 32
CH_PER_W = NCHUNK // NW
CH_EXTRA = NCHUNK - CH_PER_W * NW
CH_SUPER = (CH_PER_W // 4) * 4

BLK_E = 2000

_f32 = jnp.float32


def _worker_chunks():
    cid = lax.axis_index("c")
    sid = lax.axis_index("s")
    wid = sid * 2 + cid
    base = wid * CH_PER_W + jnp.minimum(wid, CH_EXTRA)
    n_ch = CH_PER_W + (wid < CH_EXTRA).astype(jnp.int32)
    return cid, sid, base, n_ch


GSUP = 8
GSUPN = CH_SUPER // GSUP
GSE = GSUP * 128


def _gather_body(nodes_hbm, send1d, recv1d, out_s, out_r, trash,
                 tab, ixs0, ixr0, ixs1, ixr1, rs0, rr0, rs1, rr1,
                 sem_g, sem_i, sem_w):
    cid, sid, base, n_ch = _worker_chunks()
    pltpu.sync_copy(nodes_hbm.at[pl.ds(sid * NPT, NPT)],
                    tab.at[pl.ds(sid * NPT, NPT)])
    plsc.subcore_barrier()

    ixs = (ixs0, ixs1)
    ixr = (ixr0, ixr1)
    rs = (rs0, rs1)
    rr = (rr0, rr1)

    e00 = base * 128
    pltpu.async_copy(send1d.at[pl.ds(e00, GSE)], ixs0, sem_i)
    pltpu.async_copy(recv1d.at[pl.ds(e00, GSE)], ixr0, sem_i)
    for b in (0, 1):
        pltpu.async_copy(rs[b], trash, sem_w)
        pltpu.async_copy(rr[b], trash, sem_w)

    def one_super(k, b):
        pltpu.make_async_copy(send1d.at[pl.ds(0, GSE)], ixs[b], sem_i).wait()
        pltpu.make_async_copy(recv1d.at[pl.ds(0, GSE)], ixr[b], sem_i).wait()
        pltpu.make_async_copy(trash, rs[b], sem_w).wait()
        pltpu.make_async_copy(trash, rr[b], sem_w).wait()
        for j in range(GSUP):
            sl = pl.ds(j * 128, 128)
            pltpu.async_copy(tab.at[ixs[b].at[sl]], rs[b].at[sl], sem_g)
            pltpu.async_copy(tab.at[ixr[b].at[sl]], rr[b].at[sl], sem_g)
        en = (base + jnp.minimum(k + 1, GSUPN - 1) * GSUP) * 128
        pltpu.async_copy(send1d.at[pl.ds(en, GSE)], ixs[1 - b], sem_i)
        pltpu.async_copy(recv1d.at[pl.ds(en, GSE)], ixr[1 - b], sem_i)
        for j in range(GSUP):
            sl = pl.ds(j * 128, 128)
            pltpu.make_async_copy(tab.at[ixs[b].at[sl]],
                                  rs[b].at[sl], sem_g).wait()
            pltpu.make_async_copy(tab.at[ixr[b].at[sl]],
                                  rr[b].at[sl], sem_g).wait()
        e0 = (base + k * GSUP) * 128
        pltpu.async_copy(rs[b], out_s.at[pl.ds(e0, GSE)], sem_w)
        pltpu.async_copy(rr[b], out_r.at[pl.ds(e0, GSE)], sem_w)

    def pair_body(kk, carry):
        one_super(2 * kk, 0)
        one_super(2 * kk + 1, 1)
        return carry

    lax.fori_loop(0, GSUPN // 2, pair_body, 0)

    pltpu.make_async_copy(send1d.at[pl.ds(0, GSE)], ixs0, sem_i).wait()
    pltpu.make_async_copy(recv1d.at[pl.ds(0, GSE)], ixr0, sem_i).wait()
    for b in (0, 1):
        pltpu.make_async_copy(trash, rs[b], sem_w).wait()
        pltpu.make_async_copy(trash, rr[b], sem_w).wait()

    def tail_body(t, carry):
        e0 = (base + CH_SUPER + t) * 128
        sl = pl.ds(0, 128)
        pltpu.sync_copy(send1d.at[pl.ds(e0, 128)], ixs0.at[sl])
        pltpu.sync_copy(recv1d.at[pl.ds(e0, 128)], ixr0.at[sl])
        c1 = pltpu.async_copy(tab.at[ixs0.at[sl]], rs0.at[pl.ds(0, 128)],
                              sem_g)
        c2 = pltpu.async_copy(tab.at[ixr0.at[sl]], rr0.at[pl.ds(0, 128)],
                              sem_g)
        c1.wait()
        c2.wait()
        pltpu.async_copy(rs0.at[pl.ds(0, 128)],
                         out_s.at[pl.ds(e0, 128)], sem_w).wait()
        pltpu.async_copy(rr0.at[pl.ds(0, 128)],
                         out_r.at[pl.ds(e0, 128)], sem_w).wait()
        return carry

    lax.fori_loop(0, n_ch - CH_SUPER, tail_body, 0)


def _make_gather():
    mesh = plsc.VectorSubcoreMesh(core_axis_name="c", subcore_axis_name="s")
    return pl.kernel(
        _gather_body,
        mesh=mesh,
        compiler_params=pltpu.CompilerParams(use_tc_tiling_on_sc=False),
        out_type=[jax.ShapeDtypeStruct((E, D), _f32),
                  jax.ShapeDtypeStruct((E, D), _f32),
                  jax.ShapeDtypeStruct((GSE, D), _f32)],
        scratch_types=[
            pltpu.VMEM_SHARED((NPAD, D), _f32),
            pltpu.VMEM((GSE,), jnp.int32),
            pltpu.VMEM((GSE,), jnp.int32),
            pltpu.VMEM((GSE,), jnp.int32),
            pltpu.VMEM((GSE,), jnp.int32),
            pltpu.VMEM((GSE, D), _f32),
            pltpu.VMEM((GSE, D), _f32),
            pltpu.VMEM((GSE, D), _f32),
            pltpu.VMEM((GSE, D), _f32),
            pltpu.SemaphoreType.DMA,
            pltpu.SemaphoreType.DMA,
            pltpu.SemaphoreType.DMA,
        ],
    )


def _zero_my_slice(acc, zbuf, sid):
    def zrow(i, carry):
        zbuf[i] = jnp.zeros((D,), _f32)
        return carry

    lax.fori_loop(0, 136, zrow, 0)

    def zslice(j, carry):
        pltpu.sync_copy(zbuf, acc.at[pl.ds(sid * NPT + j * 136, 136)])
        return carry

    lax.fori_loop(0, NPT // 136, zslice, 0)


SSUP = 4
SSUPN = CH_SUPER // SSUP
SSE = SSUP * 128


def _scatter_body(edges_hbm, recv2d, part_out, acc, d0, d1, ix0, ix1, zbuf,
                  sem_l):
    cid, sid, base, n_ch = _worker_chunks()
    _zero_my_slice(acc, zbuf, sid)
    plsc.subcore_barrier()

    d = (d0, d1)
    ix = (ix0, ix1)

    pltpu.async_copy(edges_hbm.at[pl.ds(base * 128, SSE)], d0, sem_l)
    pltpu.async_copy(recv2d.at[pl.ds(base, SSUP)], ix0, sem_l)

    def one_super(k, b):
        pltpu.make_async_copy(edges_hbm.at[pl.ds(0, SSE)], d[b], sem_l).wait()
        pltpu.make_async_copy(recv2d.at[pl.ds(0, SSUP)], ix[b], sem_l).wait()
        cn = base + jnp.minimum(k + 1, SSUPN - 1) * SSUP
        pltpu.async_copy(edges_hbm.at[pl.ds(cn * 128, SSE)], d[1 - b], sem_l)
        pltpu.async_copy(recv2d.at[pl.ds(cn, SSUP)], ix[1 - b], sem_l)
        for j in range(SSUP):
            pltpu.sync_copy(d[b].at[pl.ds(j * 128, 128)],
                            acc.at[ix[b].at[j]], add=True)

    def pair_body(kk, carry):
        one_super(2 * kk, 0)
        one_super(2 * kk + 1, 1)
        return carry

    lax.fori_loop(0, SSUPN // 2, pair_body, 0)
    pltpu.make_async_copy(edges_hbm.at[pl.ds(0, SSE)], d0, sem_l).wait()
    pltpu.make_async_copy(recv2d.at[pl.ds(0, SSUP)], ix0, sem_l).wait()

    def tail_body(t, carry):
        ch = base + CH_SUPER + t
        pltpu.sync_copy(edges_hbm.at[pl.ds(ch * 128, 128)],
                        d0.at[pl.ds(0, 128)])
        pltpu.sync_copy(recv2d.at[pl.ds(ch, 1)], ix0.at[pl.ds(0, 1)])
        pltpu.sync_copy(d0.at[pl.ds(0, 128)], acc.at[ix0.at[0]], add=True)
        return carry

    lax.fori_loop(0, n_ch - CH_SUPER, tail_body, 0)
    plsc.subcore_barrier()
    pltpu.sync_copy(acc.at[pl.ds(sid * NPT, NPT)],
                    part_out.at[cid, pl.ds(sid * NPT, NPT)])


def _make_scatter():
    mesh = plsc.VectorSubcoreMesh(core_axis_name="c", subcore_axis_name="s")
    return pl.kernel(
        _scatter_body,
        mesh=mesh,
        compiler_params=pltpu.CompilerParams(use_tc_tiling_on_sc=False),
        out_type=jax.ShapeDtypeStruct((2, NPAD, D), _f32),
        scratch_types=[
            pltpu.VMEM_SHARED((NPAD, D), _f32),
            pltpu.VMEM((SSE, D), _f32),
            pltpu.VMEM((SSE, D), _f32),
            pltpu.VMEM((SSUP, 128), jnp.int32),
            pltpu.VMEM((SSUP, 128), jnp.int32),
            pltpu.VMEM((136, D), _f32),
            pltpu.SemaphoreType.DMA,
        ],
    )


def _count_body(recv1d, part_out, acc, ones, idx, zbuf, sem):
    cid, sid, base, n_ch = _worker_chunks()

    def orow(i, carry):
        ones[i] = jnp.full((D,), 1.0, _f32)
        return carry

    lax.fori_loop(0, 128, orow, 0)
    _zero_my_slice(acc, zbuf, sid)
    plsc.subcore_barrier()

    def body(c, carry):
        e0 = (base + c) * 128
        pltpu.sync_copy(recv1d.at[pl.ds(e0, 128)], idx)
        pltpu.sync_copy(ones, acc.at[idx], add=True)
        return carry

    lax.fori_loop(0, n_ch, body, 0)
    plsc.subcore_barrier()
    pltpu.sync_copy(acc.at[pl.ds(sid * NPT, NPT)],
                    part_out.at[cid, pl.ds(sid * NPT, NPT)])


def _make_count():
    mesh = plsc.VectorSubcoreMesh(core_axis_name="c", subcore_axis_name="s")
    return pl.kernel(
        _count_body,
        mesh=mesh,
        compiler_params=pltpu.CompilerParams(use_tc_tiling_on_sc=False),
        out_type=jax.ShapeDtypeStruct((2, NPAD, D), _f32),
        scratch_types=[
            pltpu.VMEM_SHARED((NPAD, D), _f32),
            pltpu.VMEM((128, D), _f32),
            pltpu.VMEM((128,), jnp.int32),
            pltpu.VMEM((136, D), _f32),
            pltpu.SemaphoreType.DMA,
        ],
    )


def _edge_body(xe, xr, xs, g, w0gt, b0t, w0e, w0r, w0s, w1, b1t, w2, b2t,
               yout, esum, acc):
    i = pl.program_id(0)
    gb = jnp.dot(g[...], w0gt[...], preferred_element_type=_f32) + b0t[...]
    h = (jnp.dot(xe[...], w0e[...], preferred_element_type=_f32)
         + jnp.dot(xr[...], w0r[...], preferred_element_type=_f32)
         + jnp.dot(xs[...], w0s[...], preferred_element_type=_f32)
         + gb)
    h = jnp.maximum(h, 0.0)
    h = jnp.maximum(
        jnp.dot(h, w1[...], preferred_element_type=_f32) + b1t[...], 0.0)
    y = jnp.dot(h, w2[...], preferred_element_type=_f32) + b2t[...]
    yout[...] = y

    @pl.when(i == 0)
    def _():
        acc[...] = jnp.zeros_like(acc)

    acc[...] += jnp.sum(y, axis=0, keepdims=True)

    @pl.when(i == pl.num_programs(0) - 1)
    def _():
        esum[...] = acc[...]


def _edge_call(xe8, xr8, xs8, g, w0gt, b0t, w0e, w0r, w0s, w1, b1t, w2, b2t):
    blk = pl.BlockSpec((BLK_E, 128), lambda i: (i, 0))
    rep = lambda shape: pl.BlockSpec(shape, lambda i: (0, 0))
    return pl.pallas_call(
        _edge_body,
        grid=(E8 // BLK_E,),
        in_specs=[blk, blk, blk, rep((1, D)), rep((D, 128)), rep((1, 128)),
                  rep((128, 128)), rep((128, 128)), rep((128, 128)),
                  rep((128, 128)), rep((1, 128)),
                  rep((128, 128)), rep((1, 128))],
        out_specs=[blk, rep((1, 128))],
        out_shape=[jax.ShapeDtypeStruct((E8, 128), _f32),
                   jax.ShapeDtypeStruct((1, 128), _f32)],
        scratch_shapes=[pltpu.VMEM((1, 128), _f32)],
    )(xe8, xr8, xs8, g, w0gt, b0t, w0e, w0r, w0s, w1, b1t, w2, b2t)


def _node_body(p0, p1, c0, c1, xn, g, nw0gt, nb0t, nw0a, nw0b, nw1, nb1t,
               nw2, nb2t, esum, gw0e, gw0n, gw0g, gb0, gw1, gb1, gw2, gb2,
               yout, gout):
    cnt = c0[...] + c1[...]
    s = p0[...] + p1[...]
    inv = jnp.where(cnt > 0, 1.0 / jnp.maximum(cnt, 1.0), 0.0)
    agg = s * inv
    gb = jnp.dot(g[...], nw0gt[...], preferred_element_type=_f32) + nb0t[...]
    h = (jnp.dot(agg, nw0a[...], preferred_element_type=_f32)
         + jnp.dot(xn[...], nw0b[...], preferred_element_type=_f32)
         + gb)
    h = jnp.maximum(h, 0.0)
    h = jnp.maximum(
        jnp.dot(h, nw1[...], preferred_element_type=_f32) + nb1t[...], 0.0)
    y = jnp.dot(h, nw2[...], preferred_element_type=_f32) + nb2t[...]
    yout[...] = y

    ns = jnp.sum(y[0:N8, :], axis=0, keepdims=True)
    es = esum[...]
    e16 = jnp.zeros((1, D), _f32)
    n16 = jnp.zeros((1, D), _f32)
    for k in range(8):
        e16 = e16 + es[:, k * D:(k + 1) * D]
        n16 = n16 + ns[:, k * D:(k + 1) * D]
    mean_e = e16 * (1.0 / E)
    mean_n = n16 * (1.0 / N)
    gh = (jnp.dot(mean_e, gw0e[...], preferred_element_type=_f32)
          + jnp.dot(mean_n, gw0n[...], preferred_element_type=_f32)
          + jnp.dot(g[...], gw0g[...], preferred_element_type=_f32)
          + gb0[...])
    gh = jnp.maximum(gh, 0.0)
    gh = jnp.maximum(
        jnp.dot(gh, gw1[...], preferred_element_type=_f32) + gb1[...], 0.0)
    gout[...] = jnp.dot(gh, gw2[...], preferred_element_type=_f32) + gb2[...]


def _node_call(p0, p1, c0, c1, xn8, g, nw0gt, nb0t, nw0a, nw0b, nw1, nb1t,
               nw2, nb2t, esum, gw0e, gw0n, gw0g, gb0, gw1, gb1, gw2, gb2):
    full = lambda shape: pl.BlockSpec(shape, lambda: (0, 0))
    sds = jax.ShapeDtypeStruct
    return pl.pallas_call(
        _node_body,
        in_specs=[full((N8P, 128)), full((N8P, 128)), full((N8P, 128)),
                  full((N8P, 128)), full((N8P, 128)), full((1, D)),
                  full((D, 128)), full((1, 128)), full((128, 128)),
                  full((128, 128)), full((128, 128)), full((1, 128)),
                  full((128, 128)), full((1, 128)), full((1, 128)),
                  full((D, D)), full((D, D)), full((D, D)), full((1, D)),
                  full((D, D)), full((1, D)), full((D, D)), full((1, D))],
        out_specs=[full((N8P, 128)), full((1, D))],
        out_shape=[sds((N8P, 128), _f32), sds((1, D), _f32)],
    )(p0, p1, c0, c1, xn8, g, nw0gt, nb0t, nw0a, nw0b, nw1, nb1t, nw2, nb2t,
      esum, gw0e, gw0n, gw0g, gb0, gw1, gb1, gw2, gb2)


def kernel(nodes, edges, globals_, senders, receivers,
           ew0, eb0, ew1, eb1, ew2, eb2,
           nw0, nb0, nw1, nb1, nw2, nb2,
           gw0, gb0, gw1, gb1, gw2, gb2):
    send1d = senders.astype(jnp.int32)
    recv1d = receivers.astype(jnp.int32)
    recv2d = recv1d.reshape(NCHUNK, 128)

    i8 = jnp.eye(8, dtype=_f32)
    w0e = jnp.kron(i8, ew0[0:16])
    w0r = jnp.kron(i8, ew0[16:32])
    w0s = jnp.kron(i8, ew0[32:48])
    w0gt = jnp.tile(ew0[48:64], (1, 8))
    b0t = jnp.tile(eb0[None, :], (1, 8))
    w1k = jnp.kron(i8, ew1)
    b1t = jnp.tile(eb1[None, :], (1, 8))
    w2k = jnp.kron(i8, ew2)
    b2t = jnp.tile(eb2[None, :], (1, 8))

    nw0a = jnp.kron(i8, nw0[0:16])
    nw0b = jnp.kron(i8, nw0[16:32])
    nw0gt = jnp.tile(nw0[32:48], (1, 8))
    nb0t = jnp.tile(nb0[None, :], (1, 8))
    nw1k = jnp.kron(i8, nw1)
    nb1t = jnp.tile(nb1[None, :], (1, 8))
    nw2k = jnp.kron(i8, nw2)
    nb2t = jnp.tile(nb2[None, :], (1, 8))

    gw0e = gw0[0:16]
    gw0n = gw0[16:32]
    gw0g = gw0[32:48]
    gb0r = gb0[None, :]
    gb1r = gb1[None, :]
    gb2r = gb2[None, :]

    gather = _make_gather()
    scatter = _make_scatter()
    count = _make_count()

    cpart = count(recv1d)
    c0 = cpart[0].reshape(N8P, 128)
    c1 = cpart[1].reshape(N8P, 128)

    edges8 = edges.reshape(E8, 128)
    nodes_pad = jnp.concatenate(
        [nodes, jnp.zeros((NPAD - N, D), _f32)], axis=0)
    g_cur = globals_
    for _ in range(NPASS):
        out_s, out_r, _trash = gather(nodes_pad, send1d, recv1d)
        edges8, esum = _edge_call(
            edges8, out_r.reshape(E8, 128), out_s.reshape(E8, 128), g_cur,
            w0gt, b0t, w0e, w0r, w0s, w1k, b1t, w2k, b2t)
        part = scatter(edges8.reshape(E, D), recv2d)
        nodes8, g_cur = _node_call(
            part[0].reshape(N8P, 128), part[1].reshape(N8P, 128), c0, c1,
            nodes_pad.reshape(N8P, 128), g_cur, nw0gt, nb0t, nw0a, nw0b,
            nw1k, nb1t, nw2k, nb2t, esum,
            gw0e, gw0n, gw0g, gb0r, gw1, gb1r, gw2, gb2r)
        nodes_pad = nodes8.reshape(NPAD, D)

    return (nodes_pad[:N], edges8.reshape(E, D), g_cur)
